# baseline (speedup 1.0000x reference)
import functools

import jax
import jax.numpy as jnp
from jax import lax
from jax.experimental import pallas as pl
from jax.experimental.pallas import tpu as pltpu
from jax.experimental.pallas import tpu_sc as plsc


def _mm_body(x_ref, ws_ref, wt_ref, s0_ref, s1_ref):
    x = x_ref[...]
    s0_ref[...] = jnp.dot(x, ws_ref[...], preferred_element_type=jnp.float32)
    s1_ref[...] = jnp.dot(x, wt_ref[...], preferred_element_type=jnp.float32)


def _node_scores(x_0, att):
    n, d = x_0.shape
    h = att.shape[1]
    tm = 1000
    assert n % tm == 0
    ws = att[:d]
    wt = att[d:]
    return pl.pallas_call(
        _mm_body,
        grid=(n // tm,),
        in_specs=[
            pl.BlockSpec((tm, d), lambda i: (i, 0)),
            pl.BlockSpec((d, h), lambda i: (0, 0)),
            pl.BlockSpec((d, h), lambda i: (0, 0)),
        ],
        out_specs=[
            pl.BlockSpec((tm, h), lambda i: (i, 0)),
            pl.BlockSpec((tm, h), lambda i: (i, 0)),
        ],
        out_shape=[
            jax.ShapeDtypeStruct((n, h), jnp.float32),
            jax.ShapeDtypeStruct((n, h), jnp.float32),
        ],
    )(x_0, ws, wt)


_B = 128
_R = 8
_NW = 32


def _make_sc_edge_kernel(e_total, h):
    per_w = e_total // _NW
    n_blk = (per_w + _B - 1) // _B
    last_base = per_w - _B
    assert per_w * _NW == e_total
    assert per_w % 8 == 0 and per_w >= _B
    assert last_base % 8 == 0
    assert n_blk % _R == 0 and n_blk >= 2 * _R

    mesh = plsc.VectorSubcoreMesh(core_axis_name="c", subcore_axis_name="s")

    scratch = [
        pltpu.VMEM((per_w,), jnp.int32),
        pltpu.VMEM((per_w,), jnp.int32),
    ] + [pltpu.VMEM((_B, h), jnp.float32) for _ in range(_R)] \
      + [pltpu.VMEM((_B, h), jnp.float32) for _ in range(_R)] \
      + [pltpu.VMEM((_B * h,), jnp.float32) for _ in range(_R)] \
      + [pltpu.SemaphoreType.DMA for _ in range(2 * _R)]

    @functools.partial(
        pl.kernel,
        out_type=jax.ShapeDtypeStruct((e_total * h,), jnp.float32),
        mesh=mesh,
        scratch_types=scratch,
        compiler_params=pltpu.CompilerParams(use_tc_tiling_on_sc=False),
    )
    def sc_edge(s0, s1, src, tgt, out, idxs, idxt, *bufs):
        rows_s = bufs[0:_R]
        rows_t = bufs[_R:2 * _R]
        out_v = bufs[2 * _R:3 * _R]
        sem_g = bufs[3 * _R:3 * _R + _R]
        sem_st = bufs[4 * _R:5 * _R]

        wid = lax.axis_index("s") * 2 + lax.axis_index("c")
        base_w = wid * per_w

        pltpu.sync_copy(src.at[pl.ds(base_w, per_w)], idxs)
        pltpu.sync_copy(tgt.at[pl.ds(base_w, per_w)], idxt)

        def loc(cur):
            return jnp.minimum(cur * _B, last_base)

        def fire(cur, b):
            o = loc(cur)
            pltpu.async_copy(s0.at[idxs.at[pl.ds(o, _B)]], rows_s[b], sem_g[b])
            pltpu.async_copy(s1.at[idxt.at[pl.ds(o, _B)]], rows_t[b], sem_g[b])

        def wait_g(cur, b):
            o = loc(cur)
            pltpu.make_async_copy(
                s0.at[idxs.at[pl.ds(o, _B)]], rows_s[b], sem_g[b]).wait()
            pltpu.make_async_copy(
                s1.at[idxt.at[pl.ds(o, _B)]], rows_t[b], sem_g[b]).wait()

        def compute(b):
            for i in range(_B):
                out_v[b][pl.ds(i * h, h)] = jnp.maximum(
                    rows_s[b][i] + rows_t[b][i], 0.0)

        def store(cur, b):
            o = loc(cur)
            pltpu.async_copy(
                out_v[b], out.at[pl.ds((base_w + o) * h, _B * h)], sem_st[b])

        def wait_st(cur, b):
            o = loc(cur)
            pltpu.make_async_copy(
                out_v[b], out.at[pl.ds((base_w + o) * h, _B * h)],
                sem_st[b]).wait()

        for b in range(_R):
            fire(b, b)
        for b in range(_R):
            wait_g(b, b)
            compute(b)
            store(b, b)
            fire(b + _R, b)

        @pl.loop(_R, n_blk - _R, step=_R)
        def _steady(j):
            for b in range(_R):
                cur = j + b
                wait_g(cur, b)
                wait_st(cur - _R, b)
                compute(b)
                store(cur, b)
                fire(cur + _R, b)

        for b in range(_R):
            cur = n_blk - _R + b
            wait_g(cur, b)
            wait_st(cur - _R, b)
            compute(b)
            store(cur, b)
        for b in range(_R):
            wait_st(n_blk - _R + b, b)

    return sc_edge


def kernel(x_0, neighborhood_0_to_0, att):
    idx = neighborhood_0_to_0.astype(jnp.int32)
    e_total = idx.shape[1]
    h = att.shape[1]
    s0, s1 = _node_scores(x_0, att)
    sc_edge = _make_sc_edge_kernel(e_total, h)
    return sc_edge(s0, s1, idx[0], idx[1]).reshape(e_total, h)


# speedup vs baseline: 1.0593x; 1.0593x over previous
import functools

import jax
import jax.numpy as jnp
from jax import lax
from jax.experimental import pallas as pl
from jax.experimental.pallas import tpu as pltpu
from jax.experimental.pallas import tpu_sc as plsc


def _mm_body(x_ref, ws_ref, wt_ref, s0_ref, s1_ref):
    x = x_ref[...]
    s0_ref[...] = jnp.dot(x, ws_ref[...], preferred_element_type=jnp.float32)
    s1_ref[...] = jnp.dot(x, wt_ref[...], preferred_element_type=jnp.float32)


def _node_scores(x_0, att):
    n, d = x_0.shape
    h = att.shape[1]
    tm = 1000
    assert n % tm == 0
    ws = att[:d]
    wt = att[d:]
    return pl.pallas_call(
        _mm_body,
        grid=(n // tm,),
        in_specs=[
            pl.BlockSpec((tm, d), lambda i: (i, 0)),
            pl.BlockSpec((d, h), lambda i: (0, 0)),
            pl.BlockSpec((d, h), lambda i: (0, 0)),
        ],
        out_specs=[
            pl.BlockSpec((tm, h), lambda i: (i, 0)),
            pl.BlockSpec((tm, h), lambda i: (i, 0)),
        ],
        out_shape=[
            jax.ShapeDtypeStruct((n, h), jnp.float32),
            jax.ShapeDtypeStruct((n, h), jnp.float32),
        ],
    )(x_0, ws, wt)


_B = 128
_R = 4
_NW = 32


def _make_sc_edge_kernel(e_total, h):
    per_w = e_total // _NW
    n_blk = (per_w + _B - 1) // _B
    last_base = per_w - _B
    assert per_w * _NW == e_total
    assert per_w % 8 == 0 and per_w >= _B
    assert last_base % 8 == 0
    assert n_blk % _R == 0 and n_blk >= 2 * _R

    mesh = plsc.VectorSubcoreMesh(core_axis_name="c", subcore_axis_name="s")

    scratch = [
        pltpu.VMEM((per_w,), jnp.int32),
        pltpu.VMEM((per_w,), jnp.int32),
    ] + [pltpu.VMEM((_B, h), jnp.float32) for _ in range(_R)] \
      + [pltpu.VMEM((_B, h), jnp.float32) for _ in range(_R)] \
      + [pltpu.VMEM((_B * h,), jnp.float32) for _ in range(_R)] \
      + [pltpu.SemaphoreType.DMA for _ in range(2 * _R)]

    @functools.partial(
        pl.kernel,
        out_type=jax.ShapeDtypeStruct((e_total * h,), jnp.float32),
        mesh=mesh,
        scratch_types=scratch,
        compiler_params=pltpu.CompilerParams(use_tc_tiling_on_sc=False),
    )
    def sc_edge(s0, s1, src, tgt, out, idxs, idxt, *bufs):
        rows_s = bufs[0:_R]
        rows_t = bufs[_R:2 * _R]
        out_v = bufs[2 * _R:3 * _R]
        sem_g = bufs[3 * _R:3 * _R + _R]
        sem_st = bufs[4 * _R:5 * _R]

        wid = lax.axis_index("s") * 2 + lax.axis_index("c")
        base_w = wid * per_w

        cp_is = pltpu.async_copy(src.at[pl.ds(base_w, per_w)], idxs, sem_g[0])
        cp_it = pltpu.async_copy(tgt.at[pl.ds(base_w, per_w)], idxt, sem_g[1])
        cp_is.wait()
        cp_it.wait()

        def loc(cur):
            return jnp.minimum(cur * _B, last_base)

        def fire(cur, b):
            o = loc(cur)
            pltpu.async_copy(s0.at[idxs.at[pl.ds(o, _B)]], rows_s[b], sem_g[b])
            pltpu.async_copy(s1.at[idxt.at[pl.ds(o, _B)]], rows_t[b], sem_g[b])

        def wait_g(cur, b):
            o = loc(cur)
            pltpu.make_async_copy(
                s0.at[idxs.at[pl.ds(o, _B)]], rows_s[b], sem_g[b]).wait()
            pltpu.make_async_copy(
                s1.at[idxt.at[pl.ds(o, _B)]], rows_t[b], sem_g[b]).wait()

        def compute(b):
            for i in range(_B):
                out_v[b][pl.ds(i * h, h)] = jnp.maximum(
                    rows_s[b][i] + rows_t[b][i], 0.0)

        def store(cur, b):
            o = loc(cur)
            pltpu.async_copy(
                out_v[b], out.at[pl.ds((base_w + o) * h, _B * h)], sem_st[b])

        def wait_st(cur, b):
            o = loc(cur)
            pltpu.make_async_copy(
                out_v[b], out.at[pl.ds((base_w + o) * h, _B * h)],
                sem_st[b]).wait()

        for b in range(_R):
            fire(b, b)
        for b in range(_R):
            wait_g(b, b)
            compute(b)
            store(b, b)
            fire(b + _R, b)

        @pl.loop(_R, n_blk - _R, step=_R)
        def _steady(j):
            for b in range(_R):
                cur = j + b
                wait_g(cur, b)
                wait_st(cur - _R, b)
                compute(b)
                store(cur, b)
                fire(cur + _R, b)

        for b in range(_R):
            cur = n_blk - _R + b
            wait_g(cur, b)
            wait_st(cur - _R, b)
            compute(b)
            store(cur, b)
        for b in range(_R):
            wait_st(n_blk - _R + b, b)

    return sc_edge


def kernel(x_0, neighborhood_0_to_0, att):
    idx = neighborhood_0_to_0.astype(jnp.int32)
    e_total = idx.shape[1]
    h = att.shape[1]
    s0, s1 = _node_scores(x_0, att)
    sc_edge = _make_sc_edge_kernel(e_total, h)
    return sc_edge(s0, s1, idx[0], idx[1]).reshape(e_total, h)


# speedup vs baseline: 1.0971x; 1.0357x over previous
import functools

import jax
import jax.numpy as jnp
from jax import lax
from jax.experimental import pallas as pl
from jax.experimental.pallas import tpu as pltpu
from jax.experimental.pallas import tpu_sc as plsc


def _mm_body(x_ref, ws_ref, wt_ref, s0_ref, s1_ref):
    x = x_ref[...]
    s0_ref[...] = jnp.dot(x, ws_ref[...], preferred_element_type=jnp.float32)
    s1_ref[...] = jnp.dot(x, wt_ref[...], preferred_element_type=jnp.float32)


def _node_scores(x_0, att):
    n, d = x_0.shape
    h = att.shape[1]
    tm = 1000
    assert n % tm == 0
    ws = att[:d]
    wt = att[d:]
    return pl.pallas_call(
        _mm_body,
        grid=(n // tm,),
        in_specs=[
            pl.BlockSpec((tm, d), lambda i: (i, 0)),
            pl.BlockSpec((d, h), lambda i: (0, 0)),
            pl.BlockSpec((d, h), lambda i: (0, 0)),
        ],
        out_specs=[
            pl.BlockSpec((tm, h), lambda i: (i, 0)),
            pl.BlockSpec((tm, h), lambda i: (i, 0)),
        ],
        out_shape=[
            jax.ShapeDtypeStruct((n, h), jnp.float32),
            jax.ShapeDtypeStruct((n, h), jnp.float32),
        ],
    )(x_0, ws, wt)


_B = 128
_R = 4
_NW = 32


def _make_sc_edge_kernel(e_total, h):
    per_w = e_total // _NW
    n_blk = (per_w + _B - 1) // _B
    last_base = per_w - _B
    assert per_w * _NW == e_total
    assert per_w % 8 == 0 and per_w >= _B
    assert last_base % 8 == 0
    assert n_blk % _R == 0 and n_blk >= 2 * _R

    mesh = plsc.VectorSubcoreMesh(core_axis_name="c", subcore_axis_name="s")

    scratch = [
        pltpu.VMEM((per_w,), jnp.int32),
        pltpu.VMEM((per_w,), jnp.int32),
    ] + [pltpu.VMEM((_B, h), jnp.float32) for _ in range(_R)] \
      + [pltpu.VMEM((_B, h), jnp.float32) for _ in range(_R)] \
      + [pltpu.VMEM((_B * h,), jnp.float32) for _ in range(_R)] \
      + [pltpu.SemaphoreType.DMA for _ in range(2 * _R)]

    @functools.partial(
        pl.kernel,
        out_type=jax.ShapeDtypeStruct((e_total * h,), jnp.float32),
        mesh=mesh,
        scratch_types=scratch,
        compiler_params=pltpu.CompilerParams(use_tc_tiling_on_sc=False),
    )
    def sc_edge(s0, s1, src, tgt, out, idxs, idxt, *bufs):
        rows_s = bufs[0:_R]
        rows_t = bufs[_R:2 * _R]
        out_v = bufs[2 * _R:3 * _R]
        sem_g = bufs[3 * _R:3 * _R + _R]
        sem_st = bufs[4 * _R:5 * _R]

        wid = lax.axis_index("s") * 2 + lax.axis_index("c")
        base_w = wid * per_w

        cp_is = pltpu.async_copy(src.at[pl.ds(base_w, per_w)], idxs, sem_g[0])
        cp_it = pltpu.async_copy(tgt.at[pl.ds(base_w, per_w)], idxt, sem_g[1])
        cp_is.wait()
        cp_it.wait()

        def loc(cur):
            return jnp.minimum(cur * _B, last_base)

        def fire(cur, b):
            o = loc(cur)
            pltpu.async_copy(s0.at[idxs.at[pl.ds(o, _B)]], rows_s[b], sem_g[b])
            pltpu.async_copy(s1.at[idxt.at[pl.ds(o, _B)]], rows_t[b], sem_g[b])

        def wait_g(cur, b):
            o = loc(cur)
            pltpu.make_async_copy(
                s0.at[idxs.at[pl.ds(o, _B)]], rows_s[b], sem_g[b]).wait()
            pltpu.make_async_copy(
                s1.at[idxt.at[pl.ds(o, _B)]], rows_t[b], sem_g[b]).wait()

        def compute(b):
            @pl.loop(0, _B, step=8)
            def _rows(i):
                for u in range(8):
                    out_v[b][pl.ds((i + u) * h, h)] = jnp.maximum(
                        rows_s[b][i + u] + rows_t[b][i + u], 0.0)

        def store(cur, b):
            o = loc(cur)
            pltpu.async_copy(
                out_v[b], out.at[pl.ds((base_w + o) * h, _B * h)], sem_st[b])

        def wait_st(cur, b):
            o = loc(cur)
            pltpu.make_async_copy(
                out_v[b], out.at[pl.ds((base_w + o) * h, _B * h)],
                sem_st[b]).wait()

        for b in range(_R):
            fire(b, b)
        for b in range(_R):
            wait_g(b, b)
            compute(b)
            store(b, b)
            fire(b + _R, b)

        @pl.loop(_R, n_blk - _R, step=_R)
        def _steady(j):
            for b in range(_R):
                cur = j + b
                wait_g(cur, b)
                wait_st(cur - _R, b)
                compute(b)
                store(cur, b)
                fire(cur + _R, b)

        for b in range(_R):
            cur = n_blk - _R + b
            wait_g(cur, b)
            wait_st(cur - _R, b)
            compute(b)
            store(cur, b)
        for b in range(_R):
            wait_st(n_blk - _R + b, b)

    return sc_edge


def kernel(x_0, neighborhood_0_to_0, att):
    idx = neighborhood_0_to_0.astype(jnp.int32)
    e_total = idx.shape[1]
    h = att.shape[1]
    s0, s1 = _node_scores(x_0, att)
    sc_edge = _make_sc_edge_kernel(e_total, h)
    return sc_edge(s0, s1, idx[0], idx[1]).reshape(e_total, h)
